# Initial kernel scaffold; baseline (speedup 1.0000x reference)
import jax
import jax.numpy as jnp
from jax.experimental import pallas as pl


def kernel(rois, probs, deltas, window):
    raise NotImplementedError("write your pallas kernel here")


# speedup vs baseline: 5.5858x; 5.5858x over previous
import functools

import jax
import jax.numpy as jnp
from jax.experimental import pallas as pl
from jax.experimental.pallas import tpu as pltpu

_MIN_CONF = 0.7
_MAX_INST = 100
_NMS_THR = 0.3

_TILE = 800
_LANES = 128


def _phase_a_kernel(probs_ref, deltas_ref, s_ref, c_ref, d_ref):
    p = probs_ref[...]
    m = jnp.max(p, axis=1, keepdims=True)
    ncls = p.shape[1]
    col = jax.lax.broadcasted_iota(jnp.int32, p.shape, 1)
    cid = jnp.min(jnp.where(p == m, col, ncls), axis=1, keepdims=True)
    s_ref[...] = m
    c_ref[...] = cid.astype(jnp.float32)
    d = deltas_ref[...]
    col4 = jax.lax.broadcasted_iota(jnp.int32, d.shape, 1)
    dm = jnp.where((col4 // 4) == cid, d, 0.0)
    j = jax.lax.broadcasted_iota(jnp.int32, (d.shape[1], 4), 0)
    k = jax.lax.broadcasted_iota(jnp.int32, (d.shape[1], 4), 1)
    sel = ((j % 4) == k).astype(jnp.float32)
    d_ref[...] = jax.lax.dot_general(
        dm, sel, (((1,), (0,)), ((), ())), preferred_element_type=jnp.float32)


def _phase_b_kernel(n_total, rois_ref, dsel_ref, cid_ref, score_ref, win_ref,
                    out_ref, y1s, x1s, y2s, x2s):
    wy1 = win_ref[0]
    wx1 = win_ref[1]
    wy2 = win_ref[2]
    wx2 = win_ref[3]
    y1 = rois_ref[0]
    x1 = rois_ref[1]
    y2 = rois_ref[2]
    x2 = rois_ref[3]
    d0 = dsel_ref[0] * 0.1
    d1 = dsel_ref[1] * 0.1
    d2 = dsel_ref[2] * 0.2
    d3 = dsel_ref[3] * 0.2
    h = y2 - y1
    w = x2 - x1
    cy = y1 + 0.5 * h + d0 * h
    cx = x1 + 0.5 * w + d1 * w
    h = h * jnp.exp(d2)
    w = w * jnp.exp(d3)
    ry1 = cy - 0.5 * h
    rx1 = cx - 0.5 * w
    ry2 = ry1 + h
    rx2 = rx1 + w
    ry1 = jnp.clip(ry1, wy1, wy2)
    rx1 = jnp.clip(rx1, wx1, wx2)
    ry2 = jnp.clip(ry2, wy1, wy2)
    rx2 = jnp.clip(rx2, wx1, wx2)
    y1s[...] = ry1
    x1s[...] = rx1
    y2s[...] = ry2
    x2s[...] = rx2

    cidf = cid_ref[...]
    scr = score_ref[...]
    rowi = jax.lax.broadcasted_iota(jnp.int32, scr.shape, 0)
    coli = jax.lax.broadcasted_iota(jnp.int32, scr.shape, 1)
    flat = rowi * _LANES + coli
    in_range = flat < n_total
    keep = in_range & (cidf > 0.5) & (scr >= _MIN_CONF)
    scores0 = jnp.where(keep, scr, -1.0)

    off = cidf * 4.0
    ny1 = ry1 + off
    nx1 = rx1 + off
    ny2 = ry2 + off
    nx2 = rx2 + off
    areas = (ny2 - ny1) * (nx2 - nx1)
    lane = jax.lax.broadcasted_iota(jnp.int32, (1, _LANES), 1)
    zero_lane = jnp.zeros((1, _LANES), jnp.float32)

    def body(i, carry):
        scores, by1, bx1, by2, bx2, bcl, bsc = carry
        m = jnp.max(scores)
        idx = jnp.min(jnp.where(scores == m, flat, jnp.int32(1 << 30)))
        r = idx >> 7
        c = idx & (_LANES - 1)
        laneeq = lane == c

        def ext(ref):
            return jnp.sum(jnp.where(laneeq, ref[pl.ds(r, 1), :], 0.0))

        ey1 = ext(y1s)
        ex1 = ext(x1s)
        ey2 = ext(y2s)
        ex2 = ext(x2s)
        ecl = ext(cid_ref)
        o = ecl * 4.0
        a1 = ey1 + o
        a2 = ey2 + o
        b1 = ex1 + o
        b2 = ex2 + o
        yy1 = jnp.maximum(a1, ny1)
        xx1 = jnp.maximum(b1, nx1)
        yy2 = jnp.minimum(a2, ny2)
        xx2 = jnp.minimum(b2, nx2)
        inter = jnp.maximum(yy2 - yy1, 0.0) * jnp.maximum(xx2 - xx1, 0.0)
        union = (a2 - a1) * (b2 - b1) + areas - inter
        iou = inter / (union + 1e-8)
        supp = (iou > _NMS_THR) | (flat == idx)
        scores = jnp.where(supp, -1.0, scores)
        li = lane == i
        by1 = jnp.where(li, ey1, by1)
        bx1 = jnp.where(li, ex1, bx1)
        by2 = jnp.where(li, ey2, by2)
        bx2 = jnp.where(li, ex2, bx2)
        bcl = jnp.where(li, ecl, bcl)
        bsc = jnp.where(li, m, bsc)
        return scores, by1, bx1, by2, bx2, bcl, bsc

    init = (scores0, zero_lane, zero_lane, zero_lane, zero_lane, zero_lane,
            zero_lane)
    _, by1, bx1, by2, bx2, bcl, bsc = jax.lax.fori_loop(
        0, _MAX_INST, body, init)
    valid = bsc > 0.0
    vf = valid.astype(jnp.float32)
    out_ref[...] = jnp.concatenate(
        [by1 * vf, bx1 * vf, by2 * vf, bx2 * vf, bcl * vf,
         jnp.where(valid, bsc, 0.0), zero_lane, zero_lane], axis=0)


def kernel(rois, probs, deltas, window):
    n, ncls = probs.shape
    deltas_flat = deltas.reshape(n, ncls * 4)
    grid = n // _TILE
    s2, c2, d4 = pl.pallas_call(
        _phase_a_kernel,
        grid=(grid,),
        in_specs=[
            pl.BlockSpec((_TILE, ncls), lambda i: (i, 0)),
            pl.BlockSpec((_TILE, 4 * ncls), lambda i: (i, 0)),
        ],
        out_specs=[
            pl.BlockSpec((_TILE, 1), lambda i: (i, 0)),
            pl.BlockSpec((_TILE, 1), lambda i: (i, 0)),
            pl.BlockSpec((_TILE, 4), lambda i: (i, 0)),
        ],
        out_shape=[
            jax.ShapeDtypeStruct((n, 1), jnp.float32),
            jax.ShapeDtypeStruct((n, 1), jnp.float32),
            jax.ShapeDtypeStruct((n, 4), jnp.float32),
        ],
    )(probs, deltas_flat)

    n_pad = -(-n // _LANES) * _LANES
    rows = n_pad // _LANES
    pad = n_pad - n

    def plane(x):
        return jnp.pad(x[:, 0], (0, pad)).reshape(rows, _LANES)

    roisp = jnp.pad(rois, ((0, pad), (0, 0))).T.reshape(4, rows, _LANES)
    dselp = jnp.pad(d4, ((0, pad), (0, 0))).T.reshape(4, rows, _LANES)

    det = pl.pallas_call(
        functools.partial(_phase_b_kernel, n),
        in_specs=[
            pl.BlockSpec(memory_space=pltpu.VMEM),
            pl.BlockSpec(memory_space=pltpu.VMEM),
            pl.BlockSpec(memory_space=pltpu.VMEM),
            pl.BlockSpec(memory_space=pltpu.VMEM),
            pl.BlockSpec(memory_space=pltpu.SMEM),
        ],
        out_shape=jax.ShapeDtypeStruct((8, _LANES), jnp.float32),
        scratch_shapes=[pltpu.VMEM((rows, _LANES), jnp.float32)] * 4,
    )(roisp, dselp, plane(c2), plane(s2), window)

    boxes = det[0:4, :_MAX_INST].T
    cls = det[4:5, :_MAX_INST].T
    sc = det[5:6, :_MAX_INST].T
    return jnp.concatenate([boxes, cls, sc], axis=1)
